# baseline (speedup 1.0000x reference)
import jax
import jax.numpy as jnp
from jax import lax
from jax.experimental import pallas as pl
from jax.experimental.pallas import tpu as pltpu
from jax.experimental.pallas import tpu_sc as plsc

BATCH = 4096
DIM = 128
NUM_CORES = 2
NUM_SUBCORES = 16
NW = NUM_CORES * NUM_SUBCORES
RPW = BATCH // NW
HALF = RPW // 2
LANES = 16
CHUNKS = DIM // LANES

_MAGIC = 0x5F3759DF


def _tec_body(hs, rs, ts, ent, rel, out,
              hidx, ridx, tidx, buf_h, buf_r, buf_t, res,
              sem_i, sem0, sem1):
    cid = lax.axis_index("c")
    sid = lax.axis_index("s")
    wid = sid * NUM_CORES + cid
    base = wid * RPW

    c_h = pltpu.async_copy(hs.at[pl.ds(base, RPW)], hidx, sem_i)
    c_t = pltpu.async_copy(ts.at[pl.ds(base, RPW)], tidx, sem_i)
    c_r = pltpu.async_copy(rs.at[pl.ds(base, RPW)], ridx, sem_i)
    c_h.wait()
    c_t.wait()
    c_r.wait()

    half_sems = (sem0, sem1)

    def issue_half(k):
        lo = k * HALF
        sem = half_sems[k]
        return (
            pltpu.async_copy(ent.at[hidx.at[pl.ds(lo, HALF)]],
                             buf_h.at[pl.ds(lo, HALF)], sem),
            pltpu.async_copy(rel.at[ridx.at[pl.ds(lo, HALF)]],
                             buf_r.at[pl.ds(lo, HALF)], sem),
            pltpu.async_copy(ent.at[tidx.at[pl.ds(lo, HALF)]],
                             buf_t.at[pl.ds(lo, HALF)], sem),
        )

    lane = lax.iota(jnp.int32, LANES)
    perms = [lane ^ k for k in (1, 2, 4, 8)]

    def grp(g, _):
        y = jnp.zeros((LANES,), jnp.float32)
        for j in range(LANES):
            i = g * LANES + j
            acc = jnp.zeros((LANES,), jnp.float32)
            for c in range(CHUNKS):
                a = buf_h[i, pl.ds(c * LANES, LANES)]
                b = buf_r[i, pl.ds(c * LANES, LANES)]
                t = buf_t[i, pl.ds(c * LANES, LANES)]
                d = (a - t) + b
                acc = acc + d * d
            for p in perms:
                acc = acc + acc.at[p].get(mode="promise_in_bounds")
            y = jnp.where(lane == j, acc, y)
        ib = lax.bitcast_convert_type(y, jnp.int32)
        r = lax.bitcast_convert_type(
            _MAGIC - lax.shift_right_logical(ib, 1), jnp.float32)
        for _ in range(3):
            r = r * (1.5 - 0.5 * y * r * r)
        res[pl.ds(g * LANES, LANES)] = -(y * r)
        return 0

    g0 = issue_half(0)
    g1 = issue_half(1)
    for c in g0:
        c.wait()
    lax.fori_loop(0, HALF // LANES, grp, 0)
    for c in g1:
        c.wait()
    lax.fori_loop(HALF // LANES, RPW // LANES, grp, 0)

    pltpu.sync_copy(res, out.at[pl.ds(base, RPW)])


_mesh = plsc.VectorSubcoreMesh(core_axis_name="c", subcore_axis_name="s")

_sc_score = pl.kernel(
    _tec_body,
    out_type=jax.ShapeDtypeStruct((BATCH,), jnp.float32),
    mesh=_mesh,
    scratch_types=[
        pltpu.VMEM((RPW,), jnp.int32),
        pltpu.VMEM((RPW,), jnp.int32),
        pltpu.VMEM((RPW,), jnp.int32),
        pltpu.VMEM((RPW, DIM), jnp.float32),
        pltpu.VMEM((RPW, DIM), jnp.float32),
        pltpu.VMEM((RPW, DIM), jnp.float32),
        pltpu.VMEM((RPW,), jnp.float32),
        pltpu.SemaphoreType.DMA,
        pltpu.SemaphoreType.DMA,
        pltpu.SemaphoreType.DMA,
    ],
)


def kernel(batch, ent_embs, rel_embs):
    b = batch.astype(jnp.int32)
    hs = b[:, 0]
    rs = b[:, 1]
    ts = b[:, 2]
    score = _sc_score(hs, rs, ts, ent_embs, rel_embs)
    return score.reshape(BATCH, 1)
